# Initial kernel scaffold; baseline (speedup 1.0000x reference)
import jax
import jax.numpy as jnp
from jax.experimental import pallas as pl


def kernel(edge_index, features, W1, al1, ar1, b1, W2, al2, ar2, b2, Wfc, bfc):
    raise NotImplementedError("write your pallas kernel here")


# speedup vs baseline: 14.9248x; 14.9248x over previous
import functools

import jax
import jax.numpy as jnp
from jax import lax
from jax.experimental import pallas as pl
from jax.experimental.pallas import tpu as pltpu
from jax.experimental.pallas import tpu_sc as plsc

N = 10000
E = 320000
D = 128
H = 8
AB = 16
NC = 1
NS = 16
NW = NC * NS
EPW = E // NW
CH = 80
NCHUNK = EPW // CH
NPAD = 10112
RPS = NPAD // NS
BLK = 400
GRID = N // BLK

_f32 = jnp.float32


def _transform_tail(x, w_ref, als_ref, ald_ref, z_ref, abs_ref, abd_ref,
                    el_ref, step):
    z = jnp.dot(x, w_ref[...], preferred_element_type=_f32)
    z_ref[...] = z
    a_s = jnp.dot(z, als_ref[...], preferred_element_type=_f32)
    a_d = jnp.dot(z, ald_ref[...], preferred_element_type=_f32)
    abs_ref[...] = a_s
    abd_ref[...] = a_d
    m = jnp.max(a_s, axis=0, keepdims=True)

    @pl.when(step == 0)
    def _():
        el_ref[...] = jnp.full((1, D), -jnp.inf, _f32)

    el_ref[...] = jnp.maximum(el_ref[...], m)


def _tc1_body(x_ref, w_ref, als_ref, ald_ref,
              z_ref, abs_ref, abd_ref, el_ref):
    _transform_tail(x_ref[...], w_ref, als_ref, ald_ref,
                    z_ref, abs_ref, abd_ref, el_ref, pl.program_id(0))


def _normalize(ou_ref, den_ref, r8_ref, b_ref):
    ou = ou_ref[...]
    dn = jnp.sum(den_ref[...], axis=0)
    recip = 1.0 / (dn + 1e-9)
    rep = jnp.dot(recip, r8_ref[...], preferred_element_type=_f32)
    return ou * rep + b_ref[...]


def _tc2_body(ou_ref, den_ref, r8_ref, b_ref, w_ref, als_ref, ald_ref,
              z_ref, abs_ref, abd_ref, el_ref):
    x = _normalize(ou_ref, den_ref, r8_ref, b_ref)
    _transform_tail(x, w_ref, als_ref, ald_ref,
                    z_ref, abs_ref, abd_ref, el_ref, pl.program_id(0))


def _tc3_body(ou_ref, den_ref, r8_ref, b_ref, wfc_ref, bfc_ref, out_ref):
    x = _normalize(ou_ref, den_ref, r8_ref, b_ref)
    out_ref[...] = (jnp.dot(x, wfc_ref[...], preferred_element_type=_f32)
                    + bfc_ref[...])


_full = lambda i: (0, 0)
_rows = lambda i: (i, 0)
_rows3 = lambda i: (0, i, 0)

_transform_outs = (
    [pl.BlockSpec((BLK, D), _rows), pl.BlockSpec((BLK, D), _rows),
     pl.BlockSpec((BLK, D), _rows), pl.BlockSpec((1, D), _full)],
    [jax.ShapeDtypeStruct((N, D), _f32), jax.ShapeDtypeStruct((N, D), _f32),
     jax.ShapeDtypeStruct((N, D), _f32), jax.ShapeDtypeStruct((1, D), _f32)],
)

_tc1 = pl.pallas_call(
    _tc1_body,
    grid=(GRID,),
    in_specs=[pl.BlockSpec((BLK, D), _rows), pl.BlockSpec((D, D), _full),
              pl.BlockSpec((D, D), _full), pl.BlockSpec((D, D), _full)],
    out_specs=_transform_outs[0],
    out_shape=_transform_outs[1],
)

_norm_specs = [pl.BlockSpec((BLK, D), _rows),
               pl.BlockSpec((NS, BLK, H), _rows3),
               pl.BlockSpec((H, D), _full), pl.BlockSpec((1, D), _full)]

_tc2 = pl.pallas_call(
    _tc2_body,
    grid=(GRID,),
    in_specs=_norm_specs + [pl.BlockSpec((D, D), _full),
                            pl.BlockSpec((D, D), _full),
                            pl.BlockSpec((D, D), _full)],
    out_specs=_transform_outs[0],
    out_shape=_transform_outs[1],
)

_tc3 = pl.pallas_call(
    _tc3_body,
    grid=(GRID,),
    in_specs=_norm_specs + [pl.BlockSpec((D, D), _full),
                            pl.BlockSpec((1, D), _full)],
    out_specs=pl.BlockSpec((BLK, D), _rows),
    out_shape=jax.ShapeDtypeStruct((N, D), _f32),
)


HALF = 5056
HND = 5120
DUMP = HALF


DENW = HALF * H


def _sc_edge_body(abs_hbm, abd_hbm, z_hbm, src_hbm, dst_hbm, el_hbm,
                  zbig_hbm,
                  out_hbm, den_hbm,
                  idxs_v, idxd_v, sidx_v, abs_v, abd_v, z_v, denl_v, el_v,
                  out_sh, sem1, sem2, sem3):
    s = lax.axis_index("s")

    pltpu.sync_copy(zbig_hbm, out_sh.at[pl.ds(s * (HND // NS), HND // NS)])
    pltpu.sync_copy(el_hbm, el_v)
    plsc.subcore_barrier()

    elvec = el_v[0, pl.ds(0, 16)]
    lanes = lax.iota(jnp.int32, 16)
    msk = lanes < H
    zeros16 = jnp.zeros((16,), _f32)
    ebase = s * EPW

    def _start(c, b):
        base = ebase + c * CH
        pltpu.sync_copy(src_hbm.at[pl.ds(base, CH)], idxs_v.at[b])
        pltpu.sync_copy(dst_hbm.at[pl.ds(base, CH)], idxd_v.at[b])
        pltpu.async_copy(z_hbm.at[idxs_v.at[b]], z_v.at[b], sem3)

    def _wait(b):
        pltpu.make_async_copy(z_hbm.at[idxs_v.at[b]], z_v.at[b], sem3).wait()

    def _gather_ab(b):
        cp1 = pltpu.async_copy(abs_hbm.at[idxs_v.at[b]], abs_v, sem1)
        cp2 = pltpu.async_copy(abd_hbm.at[idxd_v.at[b]], abd_v, sem2)
        cp1.wait()
        cp2.wait()

    for p in range(2):
        lo = p * HALF

        def zero_den(j, carry2):
            denl_v[pl.ds(j * 16, 16)] = zeros16
            return carry2

        lax.fori_loop(0, (DENW + 16) // 16, zero_den, 0)

        def process(c, b):
            def group_body(j, carry2):
                d = idxd_v[b, pl.ds(j * 16, 16)] - lo
                ok = (d >= 0) & (d < HALF)
                dl16 = jnp.where(ok, d, DUMP)
                sidx_v[0, pl.ds(j * 16, 16)] = dl16

                for l in range(16):
                    i = j * 16 + l
                    kept = dl16[l] < DUMP

                    @pl.when(kept)
                    def _(i=i, dloc=dl16[l]):
                        vs = abs_v[i, pl.ds(0, 16)]
                        vd = abd_v[i, pl.ds(0, 16)]
                        t = vs + vd
                        e = jnp.where(t >= 0, t, 0.2 * t)
                        cc = vd + elvec
                        cstab = jnp.where(cc >= 0, cc, 0.2 * cc)
                        ex = jnp.exp(e - cstab)
                        plsc.addupdate_scatter(
                            denl_v, [dloc * H + lanes], ex, mask=msk)
                        for h in range(H):
                            zseg = z_v[b, i, pl.ds(h * 16, 16)]
                            z_v[b, i, pl.ds(h * 16, 16)] = zseg * ex[h]
                return carry2

            lax.fori_loop(0, CH // 16, group_body, 0)
            pltpu.sync_copy(z_v.at[b], out_sh.at[sidx_v.at[0]], add=True)

        _start(0, 0)

        def chunk_pair(k, carry):
            for b in range(2):
                c = 2 * k + b
                _wait(b)
                _start(lax.rem(c + 1, NCHUNK), 1 - b)
                _gather_ab(b)
                process(c, b)
            return carry

        lax.fori_loop(0, NCHUNK // 2, chunk_pair, 0)
        _wait(0)
        plsc.subcore_barrier()

        @pl.when(s < 8)
        def _():
            pltpu.sync_copy(out_sh.at[pl.ds(s * 632, 632)],
                            out_hbm.at[pl.ds(lo + s * 632, 632)])
        pltpu.sync_copy(denl_v.at[pl.ds(0, DENW)],
                        den_hbm.at[s, pl.ds(p * DENW, DENW)])
        plsc.subcore_barrier()
        if p == 0:
            pltpu.sync_copy(zbig_hbm,
                            out_sh.at[pl.ds(s * (HND // NS), HND // NS)])
            plsc.subcore_barrier()


_sc_edge = pl.kernel(
    _sc_edge_body,
    out_type=[jax.ShapeDtypeStruct((NPAD, D), _f32),
              jax.ShapeDtypeStruct((NS, 2 * DENW), _f32)],
    mesh=plsc.VectorSubcoreMesh(core_axis_name="c", subcore_axis_name="s",
                                num_cores=1),
    compiler_params=pltpu.CompilerParams(needs_layout_passes=False),
    scratch_types=[
        pltpu.VMEM((2, CH), jnp.int32),
        pltpu.VMEM((2, CH), jnp.int32),
        pltpu.VMEM((1, CH), jnp.int32),
        pltpu.VMEM((CH, D), _f32),
        pltpu.VMEM((CH, D), _f32),
        pltpu.VMEM((2, CH, D), _f32),
        pltpu.VMEM((DENW + 16,), _f32),
        pltpu.VMEM((1, D), _f32),
        pltpu.VMEM_SHARED((HND, D), _f32),
        pltpu.SemaphoreType.DMA,
        pltpu.SemaphoreType.DMA,
        pltpu.SemaphoreType.DMA,
    ],
)


def _pack_attn(al, ar):
    rows = jnp.arange(D)
    col = rows // 16
    m = jnp.zeros((D, D), _f32)
    return m.at[rows, col].set(al.reshape(D)).at[rows, col + H].set(
        ar.reshape(D))


def kernel(edge_index, features, W1, al1, ar1, b1, W2, al2, ar2, b2, Wfc, bfc):
    src = edge_index[0]
    dst = edge_index[1]

    als1 = _pack_attn(al1, ar1)
    ald1 = _pack_attn(ar1, al1)
    als2 = _pack_attn(al2, ar2)
    ald2 = _pack_attn(ar2, al2)

    rows = jnp.arange(D)
    r8 = jnp.zeros((H, D), _f32).at[rows // 16, rows].set(1.0)

    b1r = b1.reshape(1, D)
    b2r = b2.reshape(1, D)
    bfcr = bfc.reshape(1, D)
    zbig = jnp.zeros((HND // NS, D), _f32)

    z1, abs1, abd1, el1 = _tc1(features, W1, als1, ald1)
    ou1, den1 = _sc_edge(abs1, abd1, z1, src, dst, el1, zbig)
    z2, abs2, abd2, el2 = _tc2(ou1, den1.reshape(NS, NPAD, H), r8, b1r,
                               W2, als2, ald2)
    ou2, den2 = _sc_edge(abs2, abd2, z2, src, dst, el2, zbig)
    return _tc3(ou2, den2.reshape(NS, NPAD, H), r8, b2r, Wfc, bfcr)


# speedup vs baseline: 16.9696x; 1.1370x over previous
import functools

import jax
import jax.numpy as jnp
from jax import lax
from jax.experimental import pallas as pl
from jax.experimental.pallas import tpu as pltpu
from jax.experimental.pallas import tpu_sc as plsc

N = 10000
E = 320000
D = 128
H = 8
AB = 16
NC = 1
NS = 16
NW = NC * NS
EPW = E // NW
CH = 80
NCHUNK = EPW // CH
NPAD = 10112
RPS = NPAD // NS
BLK = 400
GRID = N // BLK

_f32 = jnp.float32


def _transform_tail(x, w_ref, als_ref, ald_ref, z_ref, abs_ref, abd_ref,
                    el_ref, step):
    z = jnp.dot(x, w_ref[...], preferred_element_type=_f32)
    z_ref[...] = z
    a_s = jnp.dot(z, als_ref[...], preferred_element_type=_f32)
    a_d = jnp.dot(z, ald_ref[...], preferred_element_type=_f32)
    abs_ref[...] = a_s
    abd_ref[...] = a_d
    m = jnp.max(a_s, axis=0, keepdims=True)

    @pl.when(step == 0)
    def _():
        el_ref[...] = jnp.full((1, D), -jnp.inf, _f32)

    el_ref[...] = jnp.maximum(el_ref[...], m)


def _tc1_body(x_ref, w_ref, als_ref, ald_ref,
              z_ref, abs_ref, abd_ref, el_ref):
    _transform_tail(x_ref[...], w_ref, als_ref, ald_ref,
                    z_ref, abs_ref, abd_ref, el_ref, pl.program_id(0))


def _normalize(ou_ref, den_ref, r8_ref, b_ref):
    ou = ou_ref[...]
    dn = jnp.sum(den_ref[...], axis=0)
    recip = 1.0 / (dn + 1e-9)
    rep = jnp.dot(recip, r8_ref[...], preferred_element_type=_f32)
    return ou * rep + b_ref[...]


def _tc2_body(ou_ref, den_ref, r8_ref, b_ref, w_ref, als_ref, ald_ref,
              z_ref, abs_ref, abd_ref, el_ref):
    x = _normalize(ou_ref, den_ref, r8_ref, b_ref)
    _transform_tail(x, w_ref, als_ref, ald_ref,
                    z_ref, abs_ref, abd_ref, el_ref, pl.program_id(0))


def _tc3_body(ou_ref, den_ref, r8_ref, b_ref, wfc_ref, bfc_ref, out_ref):
    x = _normalize(ou_ref, den_ref, r8_ref, b_ref)
    out_ref[...] = (jnp.dot(x, wfc_ref[...], preferred_element_type=_f32)
                    + bfc_ref[...])


_full = lambda i: (0, 0)
_rows = lambda i: (i, 0)
_rows3 = lambda i: (0, i, 0)

_transform_outs = (
    [pl.BlockSpec((BLK, D), _rows), pl.BlockSpec((BLK, D), _rows),
     pl.BlockSpec((BLK, D), _rows), pl.BlockSpec((1, D), _full)],
    [jax.ShapeDtypeStruct((N, D), _f32), jax.ShapeDtypeStruct((N, D), _f32),
     jax.ShapeDtypeStruct((N, D), _f32), jax.ShapeDtypeStruct((1, D), _f32)],
)

_tc1 = pl.pallas_call(
    _tc1_body,
    grid=(GRID,),
    in_specs=[pl.BlockSpec((BLK, D), _rows), pl.BlockSpec((D, D), _full),
              pl.BlockSpec((D, D), _full), pl.BlockSpec((D, D), _full)],
    out_specs=_transform_outs[0],
    out_shape=_transform_outs[1],
)

_norm_specs = [pl.BlockSpec((BLK, D), _rows),
               pl.BlockSpec((NS, BLK, H), _rows3),
               pl.BlockSpec((H, D), _full), pl.BlockSpec((1, D), _full)]

_tc2 = pl.pallas_call(
    _tc2_body,
    grid=(GRID,),
    in_specs=_norm_specs + [pl.BlockSpec((D, D), _full),
                            pl.BlockSpec((D, D), _full),
                            pl.BlockSpec((D, D), _full)],
    out_specs=_transform_outs[0],
    out_shape=_transform_outs[1],
)

_tc3 = pl.pallas_call(
    _tc3_body,
    grid=(GRID,),
    in_specs=_norm_specs + [pl.BlockSpec((D, D), _full),
                            pl.BlockSpec((1, D), _full)],
    out_specs=pl.BlockSpec((BLK, D), _rows),
    out_shape=jax.ShapeDtypeStruct((N, D), _f32),
)


HALF = 5056
HND = 5120
DUMP = HALF


DENW = HALF * H


def _sc_edge_body(abs_hbm, abd_hbm, z_hbm, src_hbm, dst_hbm, el_hbm,
                  zbig_hbm,
                  out_hbm, den_hbm,
                  idxs_v, idxd_v, sidx_v, abs_v, abd_v, z_v, denl_v, el_v,
                  out_sh, sem1, sem2, sem3):
    s = lax.axis_index("s")

    pltpu.sync_copy(zbig_hbm, out_sh.at[pl.ds(s * (HND // NS), HND // NS)])
    pltpu.sync_copy(el_hbm, el_v)
    plsc.subcore_barrier()

    elvec = el_v[0, pl.ds(0, 16)]
    lanes = lax.iota(jnp.int32, 16)
    msk = lanes < H
    zeros16 = jnp.zeros((16,), _f32)
    ebase = s * EPW

    def _start(c, b):
        base = ebase + c * CH
        pltpu.sync_copy(src_hbm.at[pl.ds(base, CH)], idxs_v.at[b])
        pltpu.sync_copy(dst_hbm.at[pl.ds(base, CH)], idxd_v.at[b])
        pltpu.async_copy(z_hbm.at[idxs_v.at[b]], z_v.at[b], sem3)

    def _wait(b):
        pltpu.make_async_copy(z_hbm.at[idxs_v.at[b]], z_v.at[b], sem3).wait()

    def _gather_ab(b):
        cp1 = pltpu.async_copy(abs_hbm.at[idxs_v.at[b]], abs_v, sem1)
        cp2 = pltpu.async_copy(abd_hbm.at[idxd_v.at[b]], abd_v, sem2)
        cp1.wait()
        cp2.wait()

    for p in range(2):
        lo = p * HALF

        def zero_den(j, carry2):
            denl_v[pl.ds(j * 16, 16)] = zeros16
            return carry2

        lax.fori_loop(0, (DENW + 16) // 16, zero_den, 0)

        def process(c, b):
            def group_body(j, carry2):
                d = idxd_v[b, pl.ds(j * 16, 16)] - lo
                ok = (d >= 0) & (d < HALF)
                dl16 = jnp.where(ok, d, DUMP)
                sidx_v[0, pl.ds(j * 16, 16)] = dl16

                for l in range(16):
                    i = j * 16 + l
                    dloc = dl16[l]
                    vs = abs_v[i, pl.ds(0, 16)]
                    vd = abd_v[i, pl.ds(0, 16)]
                    t = vs + vd
                    e = jnp.where(t >= 0, t, 0.2 * t)
                    cc = vd + elvec
                    cstab = jnp.where(cc >= 0, cc, 0.2 * cc)
                    ex = jnp.exp(e - cstab)
                    plsc.addupdate_scatter(
                        denl_v, [dloc * H + lanes], ex, mask=msk)
                    for h in range(H):
                        zseg = z_v[b, i, pl.ds(h * 16, 16)]
                        z_v[b, i, pl.ds(h * 16, 16)] = zseg * ex[h]
                return carry2

            lax.fori_loop(0, CH // 16, group_body, 0)
            pltpu.sync_copy(z_v.at[b], out_sh.at[sidx_v.at[0]], add=True)

        _start(0, 0)

        def chunk_pair(k, carry):
            for b in range(2):
                c = 2 * k + b
                _wait(b)
                _start(lax.rem(c + 1, NCHUNK), 1 - b)
                _gather_ab(b)
                process(c, b)
            return carry

        lax.fori_loop(0, NCHUNK // 2, chunk_pair, 0)
        _wait(0)
        plsc.subcore_barrier()

        @pl.when(s < 8)
        def _():
            pltpu.sync_copy(out_sh.at[pl.ds(s * 632, 632)],
                            out_hbm.at[pl.ds(lo + s * 632, 632)])
        pltpu.sync_copy(denl_v.at[pl.ds(0, DENW)],
                        den_hbm.at[s, pl.ds(p * DENW, DENW)])
        plsc.subcore_barrier()
        if p == 0:
            pltpu.sync_copy(zbig_hbm,
                            out_sh.at[pl.ds(s * (HND // NS), HND // NS)])
            plsc.subcore_barrier()


_sc_edge = pl.kernel(
    _sc_edge_body,
    out_type=[jax.ShapeDtypeStruct((NPAD, D), _f32),
              jax.ShapeDtypeStruct((NS, 2 * DENW), _f32)],
    mesh=plsc.VectorSubcoreMesh(core_axis_name="c", subcore_axis_name="s",
                                num_cores=1),
    compiler_params=pltpu.CompilerParams(needs_layout_passes=False),
    scratch_types=[
        pltpu.VMEM((2, CH), jnp.int32),
        pltpu.VMEM((2, CH), jnp.int32),
        pltpu.VMEM((1, CH), jnp.int32),
        pltpu.VMEM((CH, D), _f32),
        pltpu.VMEM((CH, D), _f32),
        pltpu.VMEM((2, CH, D), _f32),
        pltpu.VMEM((DENW + 16,), _f32),
        pltpu.VMEM((1, D), _f32),
        pltpu.VMEM_SHARED((HND, D), _f32),
        pltpu.SemaphoreType.DMA,
        pltpu.SemaphoreType.DMA,
        pltpu.SemaphoreType.DMA,
    ],
)


def _pack_attn(al, ar):
    rows = jnp.arange(D)
    col = rows // 16
    m = jnp.zeros((D, D), _f32)
    return m.at[rows, col].set(al.reshape(D)).at[rows, col + H].set(
        ar.reshape(D))


def kernel(edge_index, features, W1, al1, ar1, b1, W2, al2, ar2, b2, Wfc, bfc):
    src = edge_index[0]
    dst = edge_index[1]

    als1 = _pack_attn(al1, ar1)
    ald1 = _pack_attn(ar1, al1)
    als2 = _pack_attn(al2, ar2)
    ald2 = _pack_attn(ar2, al2)

    rows = jnp.arange(D)
    r8 = jnp.zeros((H, D), _f32).at[rows // 16, rows].set(1.0)

    b1r = b1.reshape(1, D)
    b2r = b2.reshape(1, D)
    bfcr = bfc.reshape(1, D)
    zbig = jnp.zeros((HND // NS, D), _f32)

    z1, abs1, abd1, el1 = _tc1(features, W1, als1, ald1)
    ou1, den1 = _sc_edge(abs1, abd1, z1, src, dst, el1, zbig)
    z2, abs2, abd2, el2 = _tc2(ou1, den1.reshape(NS, NPAD, H), r8, b1r,
                               W2, als2, ald2)
    ou2, den2 = _sc_edge(abs2, abd2, z2, src, dst, el2, zbig)
    return _tc3(ou2, den2.reshape(NS, NPAD, H), r8, b2r, Wfc, bfcr)


# speedup vs baseline: 21.2698x; 1.2534x over previous
import functools

import jax
import jax.numpy as jnp
from jax import lax
from jax.experimental import pallas as pl
from jax.experimental.pallas import tpu as pltpu
from jax.experimental.pallas import tpu_sc as plsc

N = 10000
E = 320000
D = 128
H = 8
AB = 16
NC = 1
NS = 16
NW = NC * NS
EPW = E // NW
CH = 80
NCHUNK = EPW // CH
NPAD = 10112
RPS = NPAD // NS
BLK = 400
GRID = N // BLK

_f32 = jnp.float32


def _transform_tail(x, w_ref, als_ref, ald_ref, z_ref, abs_ref, abd_ref,
                    el_ref, step):
    z = jnp.dot(x, w_ref[...], preferred_element_type=_f32)
    z_ref[...] = z
    a_s = jnp.dot(z, als_ref[...], preferred_element_type=_f32)
    a_d = jnp.dot(z, ald_ref[...], preferred_element_type=_f32)
    abs_ref[...] = a_s
    abd_ref[...] = a_d
    m = jnp.max(a_s, axis=0, keepdims=True)

    @pl.when(step == 0)
    def _():
        el_ref[...] = jnp.full((1, D), -jnp.inf, _f32)

    el_ref[...] = jnp.maximum(el_ref[...], m)


def _tc1_body(x_ref, w_ref, als_ref, ald_ref,
              z_ref, abs_ref, abd_ref, el_ref):
    _transform_tail(x_ref[...], w_ref, als_ref, ald_ref,
                    z_ref, abs_ref, abd_ref, el_ref, pl.program_id(0))


def _normalize(ou_ref, den_ref, r8_ref, b_ref):
    ou = ou_ref[...]
    dn = jnp.sum(den_ref[...], axis=0)
    recip = 1.0 / (dn + 1e-9)
    rep = jnp.dot(recip, r8_ref[...], preferred_element_type=_f32)
    return ou * rep + b_ref[...]


def _tc2_body(ou_ref, den_ref, r8_ref, b_ref, w_ref, als_ref, ald_ref,
              z_ref, abs_ref, abd_ref, el_ref):
    x = _normalize(ou_ref, den_ref, r8_ref, b_ref)
    _transform_tail(x, w_ref, als_ref, ald_ref,
                    z_ref, abs_ref, abd_ref, el_ref, pl.program_id(0))


def _tc3_body(ou_ref, den_ref, r8_ref, b_ref, wfc_ref, bfc_ref, out_ref):
    x = _normalize(ou_ref, den_ref, r8_ref, b_ref)
    out_ref[...] = (jnp.dot(x, wfc_ref[...], preferred_element_type=_f32)
                    + bfc_ref[...])


_full = lambda i: (0, 0)
_rows = lambda i: (i, 0)
_rows3 = lambda i: (0, i, 0)

_transform_outs = (
    [pl.BlockSpec((BLK, D), _rows), pl.BlockSpec((BLK, D), _rows),
     pl.BlockSpec((BLK, D), _rows), pl.BlockSpec((1, D), _full)],
    [jax.ShapeDtypeStruct((N, D), _f32), jax.ShapeDtypeStruct((N, D), _f32),
     jax.ShapeDtypeStruct((N, D), _f32), jax.ShapeDtypeStruct((1, D), _f32)],
)

_tc1 = pl.pallas_call(
    _tc1_body,
    grid=(GRID,),
    in_specs=[pl.BlockSpec((BLK, D), _rows), pl.BlockSpec((D, D), _full),
              pl.BlockSpec((D, D), _full), pl.BlockSpec((D, D), _full)],
    out_specs=_transform_outs[0],
    out_shape=_transform_outs[1],
)

_norm_specs = [pl.BlockSpec((BLK, D), _rows),
               pl.BlockSpec((NS, BLK, H), _rows3),
               pl.BlockSpec((H, D), _full), pl.BlockSpec((1, D), _full)]

_tc2 = pl.pallas_call(
    _tc2_body,
    grid=(GRID,),
    in_specs=_norm_specs + [pl.BlockSpec((D, D), _full),
                            pl.BlockSpec((D, D), _full),
                            pl.BlockSpec((D, D), _full)],
    out_specs=_transform_outs[0],
    out_shape=_transform_outs[1],
)

_tc3 = pl.pallas_call(
    _tc3_body,
    grid=(GRID,),
    in_specs=_norm_specs + [pl.BlockSpec((D, D), _full),
                            pl.BlockSpec((1, D), _full)],
    out_specs=pl.BlockSpec((BLK, D), _rows),
    out_shape=jax.ShapeDtypeStruct((N, D), _f32),
)


HALF = 5056
HND = 5120
DUMP = HALF


DENW = HALF * H


def _sc_edge_body(abs_hbm, abd_hbm, z_hbm, src_hbm, dst_hbm, el_hbm,
                  zbig_hbm,
                  out_hbm, den_hbm, mc_hbm, exc_hbm,
                  idxs_v, idxd_v, sidx_v, abs_v, abd_v, z_v, exw_v, exr0_v,
                  exr1_v, denl_v, el_v, out_sh, sem1, sem2, sem3):
    s = lax.axis_index("s")

    pltpu.sync_copy(zbig_hbm, out_sh.at[pl.ds(s * (HND // NS), HND // NS)])
    pltpu.sync_copy(el_hbm, el_v)
    plsc.subcore_barrier()

    elvec = el_v[0, pl.ds(0, 16)]
    lanes = lax.iota(jnp.int32, 16)
    msk = lanes < H
    zeros16 = jnp.zeros((16,), _f32)
    ebase = s * EPW

    def zero_den(j, carry2):
        denl_v[pl.ds(j * 16, 16)] = zeros16
        return carry2

    def sidx_of(b, j, lo):
        d = idxd_v[b, pl.ds(j * 16, 16)] - lo
        ok = (d >= 0) & (d < HALF)
        dl16 = jnp.where(ok, d, DUMP)
        sidx_v[0, pl.ds(j * 16, 16)] = dl16
        return dl16

    def copyout(p, lo):
        @pl.when(s < 8)
        def _():
            pltpu.sync_copy(out_sh.at[pl.ds(s * 632, 632)],
                            out_hbm.at[pl.ds(lo + s * 632, 632)])
        pltpu.sync_copy(denl_v.at[pl.ds(0, DENW)],
                        den_hbm.at[s, pl.ds(p * DENW, DENW)])
        plsc.subcore_barrier()

    lax.fori_loop(0, (DENW + 16) // 16, zero_den, 0)

    def _start(c, b):
        base = ebase + c * CH
        pltpu.sync_copy(src_hbm.at[pl.ds(base, CH)], idxs_v.at[b])
        pltpu.sync_copy(dst_hbm.at[pl.ds(base, CH)], idxd_v.at[b])
        pltpu.async_copy(z_hbm.at[idxs_v.at[b]], z_v.at[b], sem3)

    def _wait(b):
        pltpu.make_async_copy(z_hbm.at[idxs_v.at[b]], z_v.at[b], sem3).wait()

    def _gather_ab(b):
        cp1 = pltpu.async_copy(abs_hbm.at[idxs_v.at[b]], abs_v, sem1)
        cp2 = pltpu.async_copy(abd_hbm.at[idxd_v.at[b]], abd_v, sem2)
        cp1.wait()
        cp2.wait()

    def process0(c, b):
        def group_body(j, carry2):
            dl16 = sidx_of(b, j, 0)
            for l in range(16):
                i = j * 16 + l
                dloc = dl16[l]
                vs = abs_v[i, pl.ds(0, 16)]
                vd = abd_v[i, pl.ds(0, 16)]
                t = vs + vd
                e = jnp.where(t >= 0, t, 0.2 * t)
                cc = vd + elvec
                cstab = jnp.where(cc >= 0, cc, 0.2 * cc)
                ex = jnp.exp(e - cstab)
                exw_v[pl.ds(i * 16, 16)] = ex
                plsc.addupdate_scatter(
                    denl_v, [dloc * H + lanes], ex, mask=msk)
                for h in range(H):
                    zseg = z_v[b, i, pl.ds(h * 16, 16)]
                    z_v[b, i, pl.ds(h * 16, 16)] = zseg * ex[h]
            return carry2

        lax.fori_loop(0, CH // 16, group_body, 0)
        base = ebase + c * CH
        pltpu.sync_copy(z_v.at[b], out_sh.at[sidx_v.at[0]], add=True)
        pltpu.sync_copy(z_v.at[b], mc_hbm.at[pl.ds(base, CH)])
        pltpu.sync_copy(exw_v, exc_hbm.at[pl.ds(base * AB, CH * AB)])

    _start(0, 0)

    def chunk_pair0(k, carry):
        for b in range(2):
            c = 2 * k + b
            _wait(b)
            _start(lax.rem(c + 1, NCHUNK), 1 - b)
            _gather_ab(b)
            process0(c, b)
        return carry

    lax.fori_loop(0, NCHUNK // 2, chunk_pair0, 0)
    _wait(0)
    plsc.subcore_barrier()
    copyout(0, 0)
    pltpu.sync_copy(zbig_hbm, out_sh.at[pl.ds(s * (HND // NS), HND // NS)])
    plsc.subcore_barrier()

    lax.fori_loop(0, (DENW + 16) // 16, zero_den, 0)

    def _start1(c, b):
        base = ebase + c * CH
        exr = exr0_v if b == 0 else exr1_v
        pltpu.sync_copy(dst_hbm.at[pl.ds(base, CH)], idxd_v.at[b])
        pltpu.async_copy(mc_hbm.at[pl.ds(base, CH)], z_v.at[b], sem3)
        pltpu.async_copy(exc_hbm.at[pl.ds(base * AB, CH * AB)], exr, sem1)

    def _wait1(c, b):
        base = ebase + c * CH
        exr = exr0_v if b == 0 else exr1_v
        pltpu.make_async_copy(mc_hbm.at[pl.ds(base, CH)], z_v.at[b],
                              sem3).wait()
        pltpu.make_async_copy(exc_hbm.at[pl.ds(base * AB, CH * AB)], exr,
                              sem1).wait()

    def process1(c, b):
        exr = exr0_v if b == 0 else exr1_v

        def group_body(j, carry2):
            dl16 = sidx_of(b, j, HALF)
            for l in range(16):
                i = j * 16 + l
                ex = exr[pl.ds(i * 16, 16)]
                plsc.addupdate_scatter(
                    denl_v, [dl16[l] * H + lanes], ex, mask=msk)
            return carry2

        lax.fori_loop(0, CH // 16, group_body, 0)
        pltpu.sync_copy(z_v.at[b], out_sh.at[sidx_v.at[0]], add=True)

    _start1(0, 0)

    def chunk_pair1(k, carry):
        for b in range(2):
            c = 2 * k + b
            _wait1(c, b)
            _start1(lax.rem(c + 1, NCHUNK), 1 - b)
            process1(c, b)
        return carry

    lax.fori_loop(0, NCHUNK // 2, chunk_pair1, 0)
    _wait1(0, 0)
    plsc.subcore_barrier()
    copyout(1, HALF)


_sc_edge = pl.kernel(
    _sc_edge_body,
    out_type=[jax.ShapeDtypeStruct((NPAD, D), _f32),
              jax.ShapeDtypeStruct((NS, 2 * DENW), _f32),
              jax.ShapeDtypeStruct((E, D), _f32),
              jax.ShapeDtypeStruct((E * AB,), _f32)],
    mesh=plsc.VectorSubcoreMesh(core_axis_name="c", subcore_axis_name="s",
                                num_cores=1),
    compiler_params=pltpu.CompilerParams(needs_layout_passes=False),
    scratch_types=[
        pltpu.VMEM((2, CH), jnp.int32),
        pltpu.VMEM((2, CH), jnp.int32),
        pltpu.VMEM((1, CH), jnp.int32),
        pltpu.VMEM((CH, D), _f32),
        pltpu.VMEM((CH, D), _f32),
        pltpu.VMEM((2, CH, D), _f32),
        pltpu.VMEM((CH * AB,), _f32),
        pltpu.VMEM((CH * AB,), _f32),
        pltpu.VMEM((CH * AB,), _f32),
        pltpu.VMEM((DENW + 16,), _f32),
        pltpu.VMEM((1, D), _f32),
        pltpu.VMEM_SHARED((HND, D), _f32),
        pltpu.SemaphoreType.DMA,
        pltpu.SemaphoreType.DMA,
        pltpu.SemaphoreType.DMA,
    ],
)


def _pack_attn(al, ar):
    rows = jnp.arange(D)
    col = rows // 16
    m = jnp.zeros((D, D), _f32)
    return m.at[rows, col].set(al.reshape(D)).at[rows, col + H].set(
        ar.reshape(D))


def kernel(edge_index, features, W1, al1, ar1, b1, W2, al2, ar2, b2, Wfc, bfc):
    src = edge_index[0]
    dst = edge_index[1]

    als1 = _pack_attn(al1, ar1)
    ald1 = _pack_attn(ar1, al1)
    als2 = _pack_attn(al2, ar2)
    ald2 = _pack_attn(ar2, al2)

    rows = jnp.arange(D)
    r8 = jnp.zeros((H, D), _f32).at[rows // 16, rows].set(1.0)

    b1r = b1.reshape(1, D)
    b2r = b2.reshape(1, D)
    bfcr = bfc.reshape(1, D)
    zbig = jnp.zeros((HND // NS, D), _f32)

    z1, abs1, abd1, el1 = _tc1(features, W1, als1, ald1)
    ou1, den1, _, _ = _sc_edge(abs1, abd1, z1, src, dst, el1, zbig)
    z2, abs2, abd2, el2 = _tc2(ou1, den1.reshape(NS, NPAD, H), r8, b1r,
                               W2, als2, ald2)
    ou2, den2, _, _ = _sc_edge(abs2, abd2, z2, src, dst, el2, zbig)
    return _tc3(ou2, den2.reshape(NS, NPAD, H), r8, b2r, Wfc, bfcr)


# speedup vs baseline: 22.4802x; 1.0569x over previous
import functools

import jax
import jax.numpy as jnp
from jax import lax
from jax.experimental import pallas as pl
from jax.experimental.pallas import tpu as pltpu
from jax.experimental.pallas import tpu_sc as plsc

N = 10000
E = 320000
D = 128
H = 8
AB = 16
NC = 1
NS = 16
NW = NC * NS
EPW = E // NW
CH = 80
NCHUNK = EPW // CH
NPAD = 10112
RPS = NPAD // NS
BLK = 400
GRID = N // BLK

_f32 = jnp.float32


def _transform_tail(x, w_ref, als_ref, ald_ref, z_ref, abs_ref, abd_ref,
                    el_ref, step):
    z = jnp.dot(x, w_ref[...], preferred_element_type=_f32)
    z_ref[...] = z
    a_s = jnp.dot(z, als_ref[...], preferred_element_type=_f32)
    a_d = jnp.dot(z, ald_ref[...], preferred_element_type=_f32)
    abs_ref[...] = a_s
    abd_ref[...] = a_d
    m = jnp.max(a_s, axis=0, keepdims=True)

    @pl.when(step == 0)
    def _():
        el_ref[...] = jnp.full((1, D), -jnp.inf, _f32)

    el_ref[...] = jnp.maximum(el_ref[...], m)


def _tc1_body(x_ref, w_ref, als_ref, ald_ref,
              z_ref, abs_ref, abd_ref, el_ref):
    _transform_tail(x_ref[...], w_ref, als_ref, ald_ref,
                    z_ref, abs_ref, abd_ref, el_ref, pl.program_id(0))


def _normalize(ou_ref, den_ref, r8_ref, b_ref):
    ou = ou_ref[...]
    dn = jnp.sum(den_ref[...], axis=0)
    recip = 1.0 / (dn + 1e-9)
    rep = jnp.dot(recip, r8_ref[...], preferred_element_type=_f32)
    return ou * rep + b_ref[...]


def _tc2_body(ou_ref, den_ref, r8_ref, b_ref, w_ref, als_ref, ald_ref,
              z_ref, abs_ref, abd_ref, el_ref):
    x = _normalize(ou_ref, den_ref, r8_ref, b_ref)
    _transform_tail(x, w_ref, als_ref, ald_ref,
                    z_ref, abs_ref, abd_ref, el_ref, pl.program_id(0))


def _tc3_body(ou_ref, den_ref, r8_ref, b_ref, wfc_ref, bfc_ref, out_ref):
    x = _normalize(ou_ref, den_ref, r8_ref, b_ref)
    out_ref[...] = (jnp.dot(x, wfc_ref[...], preferred_element_type=_f32)
                    + bfc_ref[...])


_full = lambda i: (0, 0)
_rows = lambda i: (i, 0)
_rows3 = lambda i: (0, i, 0)

_transform_outs = (
    [pl.BlockSpec((BLK, D), _rows), pl.BlockSpec((BLK, D), _rows),
     pl.BlockSpec((BLK, D), _rows), pl.BlockSpec((1, D), _full)],
    [jax.ShapeDtypeStruct((N, D), _f32), jax.ShapeDtypeStruct((N, D), _f32),
     jax.ShapeDtypeStruct((N, D), _f32), jax.ShapeDtypeStruct((1, D), _f32)],
)

_tc1 = pl.pallas_call(
    _tc1_body,
    grid=(GRID,),
    in_specs=[pl.BlockSpec((BLK, D), _rows), pl.BlockSpec((D, D), _full),
              pl.BlockSpec((D, D), _full), pl.BlockSpec((D, D), _full)],
    out_specs=_transform_outs[0],
    out_shape=_transform_outs[1],
)

_norm_specs = [pl.BlockSpec((BLK, D), _rows),
               pl.BlockSpec((NS, BLK, H), _rows3),
               pl.BlockSpec((H, D), _full), pl.BlockSpec((1, D), _full)]

_tc2 = pl.pallas_call(
    _tc2_body,
    grid=(GRID,),
    in_specs=_norm_specs + [pl.BlockSpec((D, D), _full),
                            pl.BlockSpec((D, D), _full),
                            pl.BlockSpec((D, D), _full)],
    out_specs=_transform_outs[0],
    out_shape=_transform_outs[1],
)

_tc3 = pl.pallas_call(
    _tc3_body,
    grid=(GRID,),
    in_specs=_norm_specs + [pl.BlockSpec((D, D), _full),
                            pl.BlockSpec((1, D), _full)],
    out_specs=pl.BlockSpec((BLK, D), _rows),
    out_shape=jax.ShapeDtypeStruct((N, D), _f32),
)


HALF = 5056
HND = 5120
DUMP = HALF


DENW = HALF * H


def _sc_edge_body(abs_hbm, abd_hbm, z_hbm, src_hbm, dst_hbm, el_hbm,
                  zbig_hbm,
                  out_hbm, den_hbm, mc_hbm, exc_hbm,
                  idxs_v, idxd_v, sidx_v, abs_v, abd_v, z_v, exw0_v, exr0_v,
                  exw1_v, exr1_v, denl_v, el_v, out_sh,
                  sem1, sem2, sem3, sem4, sem5):
    s = lax.axis_index("s")

    pltpu.sync_copy(zbig_hbm, out_sh.at[pl.ds(s * (HND // NS), HND // NS)])
    pltpu.sync_copy(el_hbm, el_v)
    plsc.subcore_barrier()

    elvec = el_v[0, pl.ds(0, 16)]
    lanes = lax.iota(jnp.int32, 16)
    msk = lanes < H
    zeros16 = jnp.zeros((16,), _f32)
    ebase = s * EPW

    def zero_den(j, carry2):
        denl_v[pl.ds(j * 16, 16)] = zeros16
        return carry2

    def sidx_of(b, j, lo):
        d = idxd_v[b, pl.ds(j * 16, 16)] - lo
        ok = (d >= 0) & (d < HALF)
        dl16 = jnp.where(ok, d, DUMP)
        sidx_v[0, pl.ds(j * 16, 16)] = dl16
        return dl16

    def copyout(p, lo):
        @pl.when(s < 8)
        def _():
            pltpu.sync_copy(out_sh.at[pl.ds(s * 632, 632)],
                            out_hbm.at[pl.ds(lo + s * 632, 632)])
        pltpu.sync_copy(denl_v.at[pl.ds(0, DENW)],
                        den_hbm.at[s, pl.ds(p * DENW, DENW)])
        plsc.subcore_barrier()

    lax.fori_loop(0, (DENW + 16) // 16, zero_den, 0)

    def _start(c, b):
        base = ebase + c * CH
        pltpu.sync_copy(src_hbm.at[pl.ds(base, CH)], idxs_v.at[b])
        pltpu.sync_copy(dst_hbm.at[pl.ds(base, CH)], idxd_v.at[b])
        pltpu.async_copy(z_hbm.at[idxs_v.at[b]], z_v.at[b], sem3)

    def _wait(b):
        pltpu.make_async_copy(z_hbm.at[idxs_v.at[b]], z_v.at[b], sem3).wait()

    def _gather_ab(b):
        cp1 = pltpu.async_copy(abs_hbm.at[idxs_v.at[b]], abs_v, sem1)
        cp2 = pltpu.async_copy(abd_hbm.at[idxd_v.at[b]], abd_v, sem2)
        cp1.wait()
        cp2.wait()

    def _wait_cache(c, b):
        base = ebase + c * CH
        exw = exw0_v if b == 0 else exw1_v
        pltpu.make_async_copy(z_v.at[b], mc_hbm.at[pl.ds(base, CH)],
                              sem4).wait()
        pltpu.make_async_copy(exw, exc_hbm.at[pl.ds(base * AB, CH * AB)],
                              sem5).wait()

    def process0(c, b):
        exw_v = exw0_v if b == 0 else exw1_v

        def group_body(j, carry2):
            dl16 = sidx_of(b, j, 0)
            for l in range(16):
                i = j * 16 + l
                dloc = dl16[l]
                vs = abs_v[i, pl.ds(0, 16)]
                vd = abd_v[i, pl.ds(0, 16)]
                t = vs + vd
                e = jnp.where(t >= 0, t, 0.2 * t)
                cc = vd + elvec
                cstab = jnp.where(cc >= 0, cc, 0.2 * cc)
                ex = jnp.exp(e - cstab)
                exw_v[pl.ds(i * 16, 16)] = ex
                plsc.addupdate_scatter(
                    denl_v, [dloc * H + lanes], ex, mask=msk)
                for h in range(H):
                    zseg = z_v[b, i, pl.ds(h * 16, 16)]
                    z_v[b, i, pl.ds(h * 16, 16)] = zseg * ex[h]
            return carry2

        lax.fori_loop(0, CH // 16, group_body, 0)
        base = ebase + c * CH
        pltpu.async_copy(z_v.at[b], mc_hbm.at[pl.ds(base, CH)], sem4)
        pltpu.async_copy(exw_v, exc_hbm.at[pl.ds(base * AB, CH * AB)], sem5)
        pltpu.sync_copy(z_v.at[b], out_sh.at[sidx_v.at[0]], add=True)

    _start(0, 0)

    def chunk_pair0(k, carry):
        for b in range(2):
            c = 2 * k + b
            _wait(b)

            @pl.when(c > 0)
            def _(c=c, b=b):
                _wait_cache(c - 1, 1 - b)

            _start(lax.rem(c + 1, NCHUNK), 1 - b)
            _gather_ab(b)
            process0(c, b)
        return carry

    lax.fori_loop(0, NCHUNK // 2, chunk_pair0, 0)
    _wait(0)
    _wait_cache(NCHUNK - 1, 1)
    plsc.subcore_barrier()
    copyout(0, 0)
    pltpu.sync_copy(zbig_hbm, out_sh.at[pl.ds(s * (HND // NS), HND // NS)])
    plsc.subcore_barrier()

    lax.fori_loop(0, (DENW + 16) // 16, zero_den, 0)

    def _start1(c, b):
        base = ebase + c * CH
        exr = exr0_v if b == 0 else exr1_v
        pltpu.sync_copy(dst_hbm.at[pl.ds(base, CH)], idxd_v.at[b])
        pltpu.async_copy(mc_hbm.at[pl.ds(base, CH)], z_v.at[b], sem3)
        pltpu.async_copy(exc_hbm.at[pl.ds(base * AB, CH * AB)], exr, sem1)

    def _wait1(c, b):
        base = ebase + c * CH
        exr = exr0_v if b == 0 else exr1_v
        pltpu.make_async_copy(mc_hbm.at[pl.ds(base, CH)], z_v.at[b],
                              sem3).wait()
        pltpu.make_async_copy(exc_hbm.at[pl.ds(base * AB, CH * AB)], exr,
                              sem1).wait()

    def process1(c, b):
        exr = exr0_v if b == 0 else exr1_v

        def group_body(j, carry2):
            dl16 = sidx_of(b, j, HALF)
            for l in range(16):
                i = j * 16 + l
                ex = exr[pl.ds(i * 16, 16)]
                plsc.addupdate_scatter(
                    denl_v, [dl16[l] * H + lanes], ex, mask=msk)
            return carry2

        lax.fori_loop(0, CH // 16, group_body, 0)
        pltpu.sync_copy(z_v.at[b], out_sh.at[sidx_v.at[0]], add=True)

    _start1(0, 0)

    def chunk_pair1(k, carry):
        for b in range(2):
            c = 2 * k + b
            _wait1(c, b)
            _start1(lax.rem(c + 1, NCHUNK), 1 - b)
            process1(c, b)
        return carry

    lax.fori_loop(0, NCHUNK // 2, chunk_pair1, 0)
    _wait1(0, 0)
    plsc.subcore_barrier()
    copyout(1, HALF)


_sc_edge = pl.kernel(
    _sc_edge_body,
    out_type=[jax.ShapeDtypeStruct((NPAD, D), _f32),
              jax.ShapeDtypeStruct((NS, 2 * DENW), _f32),
              jax.ShapeDtypeStruct((E, D), _f32),
              jax.ShapeDtypeStruct((E * AB,), _f32)],
    mesh=plsc.VectorSubcoreMesh(core_axis_name="c", subcore_axis_name="s",
                                num_cores=1),
    compiler_params=pltpu.CompilerParams(needs_layout_passes=False),
    scratch_types=[
        pltpu.VMEM((2, CH), jnp.int32),
        pltpu.VMEM((2, CH), jnp.int32),
        pltpu.VMEM((1, CH), jnp.int32),
        pltpu.VMEM((CH, D), _f32),
        pltpu.VMEM((CH, D), _f32),
        pltpu.VMEM((2, CH, D), _f32),
        pltpu.VMEM((CH * AB,), _f32),
        pltpu.VMEM((CH * AB,), _f32),
        pltpu.VMEM((CH * AB,), _f32),
        pltpu.VMEM((CH * AB,), _f32),
        pltpu.VMEM((DENW + 16,), _f32),
        pltpu.VMEM((1, D), _f32),
        pltpu.VMEM_SHARED((HND, D), _f32),
        pltpu.SemaphoreType.DMA,
        pltpu.SemaphoreType.DMA,
        pltpu.SemaphoreType.DMA,
        pltpu.SemaphoreType.DMA,
        pltpu.SemaphoreType.DMA,
    ],
)


def _pack_attn(al, ar):
    rows = jnp.arange(D)
    col = rows // 16
    m = jnp.zeros((D, D), _f32)
    return m.at[rows, col].set(al.reshape(D)).at[rows, col + H].set(
        ar.reshape(D))


def kernel(edge_index, features, W1, al1, ar1, b1, W2, al2, ar2, b2, Wfc, bfc):
    src = edge_index[0]
    dst = edge_index[1]

    als1 = _pack_attn(al1, ar1)
    ald1 = _pack_attn(ar1, al1)
    als2 = _pack_attn(al2, ar2)
    ald2 = _pack_attn(ar2, al2)

    rows = jnp.arange(D)
    r8 = jnp.zeros((H, D), _f32).at[rows // 16, rows].set(1.0)

    b1r = b1.reshape(1, D)
    b2r = b2.reshape(1, D)
    bfcr = bfc.reshape(1, D)
    zbig = jnp.zeros((HND // NS, D), _f32)

    z1, abs1, abd1, el1 = _tc1(features, W1, als1, ald1)
    ou1, den1, _, _ = _sc_edge(abs1, abd1, z1, src, dst, el1, zbig)
    z2, abs2, abd2, el2 = _tc2(ou1, den1.reshape(NS, NPAD, H), r8, b1r,
                               W2, als2, ald2)
    ou2, den2, _, _ = _sc_edge(abs2, abd2, z2, src, dst, el2, zbig)
    return _tc3(ou2, den2.reshape(NS, NPAD, H), r8, b2r, Wfc, bfcr)


# speedup vs baseline: 22.4961x; 1.0007x over previous
import functools

import jax
import jax.numpy as jnp
from jax import lax
from jax.experimental import pallas as pl
from jax.experimental.pallas import tpu as pltpu
from jax.experimental.pallas import tpu_sc as plsc

N = 10000
E = 320000
D = 128
H = 8
AB = 16
NC = 1
NS = 16
NW = NC * NS
EPW = E // NW
CH = 80
NCHUNK = EPW // CH
NPAD = 10112
RPS = NPAD // NS
BLK = 400
GRID = N // BLK

_f32 = jnp.float32


def _transform_tail(x, w_ref, als_ref, ald_ref, z_ref, abs_ref, abd_ref,
                    el_ref, step):
    z = jnp.dot(x, w_ref[...], preferred_element_type=_f32)
    z_ref[...] = z
    a_s = jnp.dot(z, als_ref[...], preferred_element_type=_f32)
    a_d = jnp.dot(z, ald_ref[...], preferred_element_type=_f32)
    abs_ref[...] = a_s
    abd_ref[...] = a_d
    m = jnp.max(a_s, axis=0, keepdims=True)

    @pl.when(step == 0)
    def _():
        el_ref[...] = jnp.full((1, D), -jnp.inf, _f32)

    el_ref[...] = jnp.maximum(el_ref[...], m)


def _tc1_body(x_ref, w_ref, als_ref, ald_ref,
              z_ref, abs_ref, abd_ref, el_ref):
    _transform_tail(x_ref[...], w_ref, als_ref, ald_ref,
                    z_ref, abs_ref, abd_ref, el_ref, pl.program_id(0))


def _normalize(ou_ref, den_ref, r8_ref, b_ref):
    ou = ou_ref[...]
    dn = jnp.sum(den_ref[...], axis=0)
    recip = 1.0 / (dn + 1e-9)
    rep = jnp.dot(recip, r8_ref[...], preferred_element_type=_f32)
    return ou * rep + b_ref[...]


def _tc2_body(ou_ref, den_ref, r8_ref, b_ref, w_ref, als_ref, ald_ref,
              z_ref, abs_ref, abd_ref, el_ref):
    x = _normalize(ou_ref, den_ref, r8_ref, b_ref)
    _transform_tail(x, w_ref, als_ref, ald_ref,
                    z_ref, abs_ref, abd_ref, el_ref, pl.program_id(0))


def _tc3_body(ou_ref, den_ref, r8_ref, b_ref, wfc_ref, bfc_ref, out_ref):
    x = _normalize(ou_ref, den_ref, r8_ref, b_ref)
    out_ref[...] = (jnp.dot(x, wfc_ref[...], preferred_element_type=_f32)
                    + bfc_ref[...])


_full = lambda i: (0, 0)
_rows = lambda i: (i, 0)
_rows3 = lambda i: (0, i, 0)

_transform_outs = (
    [pl.BlockSpec((BLK, D), _rows), pl.BlockSpec((BLK, D), _rows),
     pl.BlockSpec((BLK, D), _rows), pl.BlockSpec((1, D), _full)],
    [jax.ShapeDtypeStruct((N, D), _f32), jax.ShapeDtypeStruct((N, D), _f32),
     jax.ShapeDtypeStruct((N, D), _f32), jax.ShapeDtypeStruct((1, D), _f32)],
)

_tc1 = pl.pallas_call(
    _tc1_body,
    grid=(GRID,),
    in_specs=[pl.BlockSpec((BLK, D), _rows), pl.BlockSpec((D, D), _full),
              pl.BlockSpec((D, D), _full), pl.BlockSpec((D, D), _full)],
    out_specs=_transform_outs[0],
    out_shape=_transform_outs[1],
)

_norm_specs = [pl.BlockSpec((BLK, D), _rows),
               pl.BlockSpec((NS, BLK, H), _rows3),
               pl.BlockSpec((H, D), _full), pl.BlockSpec((1, D), _full)]

_tc2 = pl.pallas_call(
    _tc2_body,
    grid=(GRID,),
    in_specs=_norm_specs + [pl.BlockSpec((D, D), _full),
                            pl.BlockSpec((D, D), _full),
                            pl.BlockSpec((D, D), _full)],
    out_specs=_transform_outs[0],
    out_shape=_transform_outs[1],
)

_tc3 = pl.pallas_call(
    _tc3_body,
    grid=(GRID,),
    in_specs=_norm_specs + [pl.BlockSpec((D, D), _full),
                            pl.BlockSpec((1, D), _full)],
    out_specs=pl.BlockSpec((BLK, D), _rows),
    out_shape=jax.ShapeDtypeStruct((N, D), _f32),
)


HALF = 5056
HND = 5120
DUMP = HALF


DENW = HALF * H


def _sc_edge_body(abs_hbm, abd_hbm, z_hbm, src_hbm, dst_hbm, el_hbm,
                  zbig_hbm,
                  out_hbm, den_hbm, mc_hbm, exc_hbm,
                  idxs_v, idxd_v, sidx_v, abs_v, abd_v, z_v, exw0_v, exr0_v,
                  exw1_v, exr1_v, denl_v, el_v, out_sh,
                  sem1, sem2, sem3, sem4, sem5, sem6):
    s = lax.axis_index("s")

    pltpu.sync_copy(zbig_hbm, out_sh.at[pl.ds(s * (HND // NS), HND // NS)])
    pltpu.sync_copy(el_hbm, el_v)
    plsc.subcore_barrier()

    elvec = el_v[0, pl.ds(0, 16)]
    lanes = lax.iota(jnp.int32, 16)
    msk = lanes < H
    zeros16 = jnp.zeros((16,), _f32)
    ebase = s * EPW

    def zero_den(j, carry2):
        denl_v[pl.ds(j * 16, 16)] = zeros16
        return carry2

    def sidx_of(b, j, lo):
        d = idxd_v[b, pl.ds(j * 16, 16)] - lo
        ok = (d >= 0) & (d < HALF)
        dl16 = jnp.where(ok, d, DUMP)
        sidx_v[b, pl.ds(j * 16, 16)] = dl16
        return dl16

    def _wait_scatter(b):
        pltpu.make_async_copy(z_v.at[b], out_sh.at[sidx_v.at[b]],
                              sem6).wait()

    def copyout(p, lo):
        @pl.when(s < 8)
        def _():
            pltpu.sync_copy(out_sh.at[pl.ds(s * 632, 632)],
                            out_hbm.at[pl.ds(lo + s * 632, 632)])
        pltpu.sync_copy(denl_v.at[pl.ds(0, DENW)],
                        den_hbm.at[s, pl.ds(p * DENW, DENW)])
        plsc.subcore_barrier()

    lax.fori_loop(0, (DENW + 16) // 16, zero_den, 0)

    def _start(c, b):
        base = ebase + c * CH
        pltpu.sync_copy(src_hbm.at[pl.ds(base, CH)], idxs_v.at[b])
        pltpu.sync_copy(dst_hbm.at[pl.ds(base, CH)], idxd_v.at[b])
        pltpu.async_copy(z_hbm.at[idxs_v.at[b]], z_v.at[b], sem3)

    def _wait(b):
        pltpu.make_async_copy(z_hbm.at[idxs_v.at[b]], z_v.at[b], sem3).wait()

    def _gather_ab(b):
        cp1 = pltpu.async_copy(abs_hbm.at[idxs_v.at[b]], abs_v, sem1)
        cp2 = pltpu.async_copy(abd_hbm.at[idxd_v.at[b]], abd_v, sem2)
        cp1.wait()
        cp2.wait()

    def _wait_cache(c, b):
        base = ebase + c * CH
        exw = exw0_v if b == 0 else exw1_v
        pltpu.make_async_copy(z_v.at[b], mc_hbm.at[pl.ds(base, CH)],
                              sem4).wait()
        pltpu.make_async_copy(exw, exc_hbm.at[pl.ds(base * AB, CH * AB)],
                              sem5).wait()

    def process0(c, b):
        exw_v = exw0_v if b == 0 else exw1_v

        def group_body(j, carry2):
            dl16 = sidx_of(b, j, 0)
            for l in range(16):
                i = j * 16 + l
                dloc = dl16[l]
                vs = abs_v[i, pl.ds(0, 16)]
                vd = abd_v[i, pl.ds(0, 16)]
                t = vs + vd
                e = jnp.where(t >= 0, t, 0.2 * t)
                cc = vd + elvec
                cstab = jnp.where(cc >= 0, cc, 0.2 * cc)
                ex = jnp.exp(e - cstab)
                exw_v[pl.ds(i * 16, 16)] = ex
                plsc.addupdate_scatter(
                    denl_v, [dloc * H + lanes], ex, mask=msk)
                for h in range(H):
                    zseg = z_v[b, i, pl.ds(h * 16, 16)]
                    z_v[b, i, pl.ds(h * 16, 16)] = zseg * ex[h]
            return carry2

        lax.fori_loop(0, CH // 16, group_body, 0)
        base = ebase + c * CH
        pltpu.async_copy(z_v.at[b], mc_hbm.at[pl.ds(base, CH)], sem4)
        pltpu.async_copy(exw_v, exc_hbm.at[pl.ds(base * AB, CH * AB)], sem5)
        pltpu.async_copy(z_v.at[b], out_sh.at[sidx_v.at[b]], sem6, add=True)

    _start(0, 0)

    def chunk_pair0(k, carry):
        for b in range(2):
            c = 2 * k + b
            _wait(b)

            @pl.when(c > 0)
            def _(c=c, b=b):
                _wait_cache(c - 1, 1 - b)
                _wait_scatter(1 - b)

            _start(lax.rem(c + 1, NCHUNK), 1 - b)
            _gather_ab(b)
            process0(c, b)
        return carry

    lax.fori_loop(0, NCHUNK // 2, chunk_pair0, 0)
    _wait(0)
    _wait_cache(NCHUNK - 1, 1)
    _wait_scatter(1)
    plsc.subcore_barrier()
    copyout(0, 0)
    pltpu.sync_copy(zbig_hbm, out_sh.at[pl.ds(s * (HND // NS), HND // NS)])
    plsc.subcore_barrier()

    lax.fori_loop(0, (DENW + 16) // 16, zero_den, 0)

    def _start1(c, b):
        base = ebase + c * CH
        exr = exr0_v if b == 0 else exr1_v
        pltpu.sync_copy(dst_hbm.at[pl.ds(base, CH)], idxd_v.at[b])
        pltpu.async_copy(mc_hbm.at[pl.ds(base, CH)], z_v.at[b], sem3)
        pltpu.async_copy(exc_hbm.at[pl.ds(base * AB, CH * AB)], exr, sem1)

    def _wait1(c, b):
        base = ebase + c * CH
        exr = exr0_v if b == 0 else exr1_v
        pltpu.make_async_copy(mc_hbm.at[pl.ds(base, CH)], z_v.at[b],
                              sem3).wait()
        pltpu.make_async_copy(exc_hbm.at[pl.ds(base * AB, CH * AB)], exr,
                              sem1).wait()

    def process1(c, b):
        exr = exr0_v if b == 0 else exr1_v

        def group_body(j, carry2):
            dl16 = sidx_of(b, j, HALF)
            for l in range(16):
                i = j * 16 + l
                ex = exr[pl.ds(i * 16, 16)]
                plsc.addupdate_scatter(
                    denl_v, [dl16[l] * H + lanes], ex, mask=msk)
            return carry2

        lax.fori_loop(0, CH // 16, group_body, 0)
        pltpu.async_copy(z_v.at[b], out_sh.at[sidx_v.at[b]], sem6, add=True)

    _start1(0, 0)

    def chunk_pair1(k, carry):
        for b in range(2):
            c = 2 * k + b
            _wait1(c, b)

            @pl.when(c > 0)
            def _(b=b):
                _wait_scatter(1 - b)

            _start1(lax.rem(c + 1, NCHUNK), 1 - b)
            process1(c, b)
        return carry

    lax.fori_loop(0, NCHUNK // 2, chunk_pair1, 0)
    _wait1(0, 0)
    _wait_scatter(1)
    plsc.subcore_barrier()
    copyout(1, HALF)


_sc_edge = pl.kernel(
    _sc_edge_body,
    out_type=[jax.ShapeDtypeStruct((NPAD, D), _f32),
              jax.ShapeDtypeStruct((NS, 2 * DENW), _f32),
              jax.ShapeDtypeStruct((E, D), _f32),
              jax.ShapeDtypeStruct((E * AB,), _f32)],
    mesh=plsc.VectorSubcoreMesh(core_axis_name="c", subcore_axis_name="s",
                                num_cores=1),
    compiler_params=pltpu.CompilerParams(needs_layout_passes=False),
    scratch_types=[
        pltpu.VMEM((2, CH), jnp.int32),
        pltpu.VMEM((2, CH), jnp.int32),
        pltpu.VMEM((2, CH), jnp.int32),
        pltpu.VMEM((CH, D), _f32),
        pltpu.VMEM((CH, D), _f32),
        pltpu.VMEM((2, CH, D), _f32),
        pltpu.VMEM((CH * AB,), _f32),
        pltpu.VMEM((CH * AB,), _f32),
        pltpu.VMEM((CH * AB,), _f32),
        pltpu.VMEM((CH * AB,), _f32),
        pltpu.VMEM((DENW + 16,), _f32),
        pltpu.VMEM((1, D), _f32),
        pltpu.VMEM_SHARED((HND, D), _f32),
        pltpu.SemaphoreType.DMA,
        pltpu.SemaphoreType.DMA,
        pltpu.SemaphoreType.DMA,
        pltpu.SemaphoreType.DMA,
        pltpu.SemaphoreType.DMA,
        pltpu.SemaphoreType.DMA,
    ],
)


def _pack_attn(al, ar):
    rows = jnp.arange(D)
    col = rows // 16
    m = jnp.zeros((D, D), _f32)
    return m.at[rows, col].set(al.reshape(D)).at[rows, col + H].set(
        ar.reshape(D))


def kernel(edge_index, features, W1, al1, ar1, b1, W2, al2, ar2, b2, Wfc, bfc):
    src = edge_index[0]
    dst = edge_index[1]

    als1 = _pack_attn(al1, ar1)
    ald1 = _pack_attn(ar1, al1)
    als2 = _pack_attn(al2, ar2)
    ald2 = _pack_attn(ar2, al2)

    rows = jnp.arange(D)
    r8 = jnp.zeros((H, D), _f32).at[rows // 16, rows].set(1.0)

    b1r = b1.reshape(1, D)
    b2r = b2.reshape(1, D)
    bfcr = bfc.reshape(1, D)
    zbig = jnp.zeros((HND // NS, D), _f32)

    z1, abs1, abd1, el1 = _tc1(features, W1, als1, ald1)
    ou1, den1, _, _ = _sc_edge(abs1, abd1, z1, src, dst, el1, zbig)
    z2, abs2, abd2, el2 = _tc2(ou1, den1.reshape(NS, NPAD, H), r8, b1r,
                               W2, als2, ald2)
    ou2, den2, _, _ = _sc_edge(abs2, abd2, z2, src, dst, el2, zbig)
    return _tc3(ou2, den2.reshape(NS, NPAD, H), r8, b2r, Wfc, bfcr)
